# Initial kernel scaffold; baseline (speedup 1.0000x reference)
import jax
import jax.numpy as jnp
from jax.experimental import pallas as pl


def kernel(points, point_features, params):
    raise NotImplementedError("write your pallas kernel here")


# speedup vs baseline: 8.3781x; 8.3781x over previous
import functools

import jax
import jax.numpy as jnp
from jax import lax
from jax.experimental import pallas as pl
from jax.experimental.pallas import tpu as pltpu
from jax.experimental.pallas import tpu_sc as plsc


def _fps_body(npoint, N, B, xyz_ref, cx_ref, cy_ref, cz_ref):
    xr = xyz_ref[:, 0, :]
    yr = xyz_ref[:, 1, :]
    zr = xyz_ref[:, 2, :]
    lane = lax.broadcasted_iota(jnp.int32, (B, N), 1)
    col = lax.broadcasted_iota(jnp.int32, (B, npoint), 1)

    def body(i, c):
        dist, far, cx_a, cy_a, cz_a = c
        oh = lane == far
        cx = jnp.sum(jnp.where(oh, xr, 0.0), axis=1, keepdims=True)
        cy = jnp.sum(jnp.where(oh, yr, 0.0), axis=1, keepdims=True)
        cz = jnp.sum(jnp.where(oh, zr, 0.0), axis=1, keepdims=True)
        sel = col == i
        cx_a = jnp.where(sel, cx, cx_a)
        cy_a = jnp.where(sel, cy, cy_a)
        cz_a = jnp.where(sel, cz, cz_a)
        dx = xr - cx
        dy = yr - cy
        dz = zr - cz
        d = dx * dx + dy * dy + dz * dz
        dist = jnp.minimum(dist, d)
        mx = jnp.max(dist, axis=1, keepdims=True)
        far = jnp.min(jnp.where(dist == mx, lane, N), axis=1, keepdims=True)
        return (dist, far, cx_a, cy_a, cz_a)

    init = (
        jnp.full((B, N), 1e10, jnp.float32),
        jnp.zeros((B, 1), jnp.int32),
        jnp.zeros((B, npoint), jnp.float32),
        jnp.zeros((B, npoint), jnp.float32),
        jnp.zeros((B, npoint), jnp.float32),
    )
    _, _, cx_a, cy_a, cz_a = lax.fori_loop(0, npoint, body, init)
    cx_ref[...] = cx_a
    cy_ref[...] = cy_a
    cz_ref[...] = cz_a


def _fps(xyz_t, npoint):
    B, _, N = xyz_t.shape
    out = jax.ShapeDtypeStruct((B, npoint), jnp.float32)
    return pl.pallas_call(
        functools.partial(_fps_body, npoint, N, B),
        out_shape=(out, out, out),
    )(xyz_t)


def _round_bf16(x):
    u = lax.bitcast_convert_type(x, jnp.uint32)
    r = (u + 0x7FFF + ((u >> 16) & 1)) & jnp.uint32(0xFFFF0000)
    return lax.bitcast_convert_type(r, jnp.float32)


def _bq_body(r2, K, N, S_T, nxs_ref, xyz_ref, out_ref):
    b = pl.program_id(0)
    sx = nxs_ref[0, :, 0:1]
    sy = nxs_ref[0, :, 1:2]
    sz = nxs_ref[0, :, 2:3]
    nx = xyz_ref[0, 0:1, :]
    ny = xyz_ref[0, 1:2, :]
    nz = xyz_ref[0, 2:3, :]
    dots = (
        _round_bf16(sx) * _round_bf16(nx)
        + _round_bf16(sy) * _round_bf16(ny)
        + _round_bf16(sz) * _round_bf16(nz)
    )
    s2 = sx * sx + sy * sy + sz * sz
    n2 = nx * nx + ny * ny + nz * nz
    sqd = (s2 + n2) - 2.0 * dots
    lane = lax.broadcasted_iota(jnp.int32, (S_T, N), 1)
    colk = lax.broadcasted_iota(jnp.int32, (S_T, K), 1)
    masked = jnp.where(sqd <= r2, lane, N)

    def step(j, c):
        masked, first, out = c
        m = jnp.min(masked, axis=1, keepdims=True)
        first = jnp.where(j == 0, m, first)
        sel = jnp.where(m == N, first, m)
        out = jnp.where(colk == j, sel, out)
        masked = jnp.where(masked == m, N, masked)
        return (masked, first, out)

    init = (masked, jnp.zeros((S_T, 1), jnp.int32), jnp.zeros((S_T, K), jnp.int32))
    _, _, out = lax.fori_loop(0, K, step, init)
    out = jnp.minimum(out, N - 1)
    out_ref[...] = (out + b * N)[None, :, :]


def _ballq(nxs, xyz_t, radius, K):
    B, S, _ = nxs.shape
    N = xyz_t.shape[2]
    S_T = 64
    return pl.pallas_call(
        functools.partial(_bq_body, radius * radius, K, N, S_T),
        grid=(B, S // S_T),
        in_specs=[
            pl.BlockSpec((1, S_T, 3), lambda b, s: (b, s, 0)),
            pl.BlockSpec((1, 3, N), lambda b, s: (b, 0, 0)),
        ],
        out_specs=pl.BlockSpec((1, S_T, K), lambda b, s: (b, s, 0)),
        out_shape=jax.ShapeDtypeStruct((B, S, K), jnp.int32),
    )(nxs, xyz_t)


def _sc_gather(table, idx):
    V, D = table.shape
    R = idx.shape[0]
    NW = 32
    rpw = R // NW
    nch = rpw // 128
    mesh = plsc.VectorSubcoreMesh(core_axis_name="c", subcore_axis_name="s")

    @functools.partial(
        pl.kernel,
        out_type=jax.ShapeDtypeStruct((R, D), jnp.float32),
        mesh=mesh,
        compiler_params=pltpu.CompilerParams(use_tc_tiling_on_sc=False),
        scratch_types=[
            pltpu.VMEM((rpw,), jnp.int32),
            pltpu.VMEM((128, D), jnp.float32),
            pltpu.SemaphoreType.DMA,
        ],
    )
    def k(table_hbm, idx_hbm, out_hbm, idx_v, buf_v, sem):
        wid = lax.axis_index("s") * 2 + lax.axis_index("c")
        base = wid * rpw
        pltpu.sync_copy(idx_hbm.at[pl.ds(base, rpw)], idx_v)

        def body(i, _):
            pltpu.async_copy(
                table_hbm.at[idx_v.at[pl.ds(i * 128, 128)]], buf_v, sem
            ).wait()
            pltpu.sync_copy(buf_v, out_hbm.at[pl.ds(base + i * 128, 128)])
            return 0

        lax.fori_loop(0, nch, body, 0)

    return k(table, idx)


def _mlpA_body(K, nxp_ref, g_ref, w_ref, bias_ref, y_ref, st_ref):
    R_T = g_ref.shape[0]
    G_T = R_T // K
    x = g_ref[...].reshape(G_T, K, -1) - nxp_ref[...][:, None, :]
    x = x.reshape(R_T, -1)
    y = jnp.dot(x, w_ref[...], preferred_element_type=jnp.float32) + bias_ref[0:1, :]
    y_ref[...] = y

    @pl.when(pl.program_id(0) == 0)
    def _():
        st_ref[...] = jnp.zeros_like(st_ref)

    st_ref[0:1, :] += jnp.sum(y, axis=0, keepdims=True)
    st_ref[1:2, :] += jnp.sum(y * y, axis=0, keepdims=True)


def _mlpB_body(aff_ref, y_ref, w_ref, bias_ref, y2_ref, st_ref):
    z = jnp.maximum(y_ref[...] * aff_ref[0:1, :] + aff_ref[1:2, :], 0.0)
    y2 = jnp.dot(z, w_ref[...], preferred_element_type=jnp.float32) + bias_ref[0:1, :]
    y2_ref[...] = y2

    @pl.when(pl.program_id(0) == 0)
    def _():
        st_ref[...] = jnp.zeros_like(st_ref)

    st_ref[0:1, :] += jnp.sum(y2, axis=0, keepdims=True)
    st_ref[1:2, :] += jnp.sum(y2 * y2, axis=0, keepdims=True)


def _mlpD_body(K, aff_ref, y_ref, o_ref):
    z = jnp.maximum(y_ref[...] * aff_ref[0:1, :] + aff_ref[1:2, :], 0.0)
    R_T = z.shape[0]
    o_ref[...] = jnp.max(z.reshape(R_T // K, K, -1), axis=1)


def _affine(st, gamma, beta, count):
    mean = st[0] / count
    var = st[1] / count - mean * mean
    scale = gamma / jnp.sqrt(var + 1e-5)
    shift = beta - mean * scale
    z = jnp.zeros_like(scale)
    return jnp.stack([scale, shift, z, z, z, z, z, z])


def _row8(v):
    return jnp.concatenate([v[None, :], jnp.zeros((7, v.shape[0]), v.dtype)])


def _mlp_sa(g, nxs, lp, K, Dp, R_T):
    R = g.shape[0]
    G = R // K
    G_T = R_T // K
    grid = (R // R_T,)
    (W1, b1, ga1, be1), (W2, b2, ga2, be2), (W3, b3, ga3, be3) = lp
    C1, C2, C3 = W1.shape[0], W2.shape[0], W3.shape[0]
    cnt = jnp.float32(R)

    W1p = jnp.zeros((Dp, C1), jnp.float32).at[: W1.shape[1], :].set(W1.T)
    nxp = jnp.zeros((G, Dp), jnp.float32).at[:, :3].set(nxs)

    def statspec(C):
        return pl.BlockSpec((8, C), lambda i: (0, 0))

    y1, st1 = pl.pallas_call(
        functools.partial(_mlpA_body, K),
        grid=grid,
        in_specs=[
            pl.BlockSpec((G_T, Dp), lambda i: (i, 0)),
            pl.BlockSpec((R_T, Dp), lambda i: (i, 0)),
            pl.BlockSpec((Dp, C1), lambda i: (0, 0)),
            statspec(C1),
        ],
        out_specs=[pl.BlockSpec((R_T, C1), lambda i: (i, 0)), statspec(C1)],
        out_shape=[
            jax.ShapeDtypeStruct((R, C1), jnp.float32),
            jax.ShapeDtypeStruct((8, C1), jnp.float32),
        ],
    )(nxp, g, W1p, _row8(b1))

    def stage_b(aff, y, W, b, Cin, Cout):
        return pl.pallas_call(
            _mlpB_body,
            grid=grid,
            in_specs=[
                statspec(Cin),
                pl.BlockSpec((R_T, Cin), lambda i: (i, 0)),
                pl.BlockSpec((Cin, Cout), lambda i: (0, 0)),
                statspec(Cout),
            ],
            out_specs=[pl.BlockSpec((R_T, Cout), lambda i: (i, 0)), statspec(Cout)],
            out_shape=[
                jax.ShapeDtypeStruct((R, Cout), jnp.float32),
                jax.ShapeDtypeStruct((8, Cout), jnp.float32),
            ],
        )(aff, y, W.T, _row8(b))

    y2, st2 = stage_b(_affine(st1, ga1, be1, cnt), y1, W2, b2, C1, C2)
    y3, st3 = stage_b(_affine(st2, ga2, be2, cnt), y2, W3, b3, C2, C3)

    out = pl.pallas_call(
        functools.partial(_mlpD_body, K),
        grid=grid,
        in_specs=[
            statspec(C3),
            pl.BlockSpec((R_T, C3), lambda i: (i, 0)),
        ],
        out_specs=pl.BlockSpec((G_T, C3), lambda i: (i, 0)),
        out_shape=jax.ShapeDtypeStruct((G, C3), jnp.float32),
    )(_affine(st3, ga3, be3, cnt), y3)
    return out


def _sa3_body(B, S, x_ref, w1_ref, p1_ref, w2_ref, p2_ref, w3_ref, p3_ref, o_ref):
    x = x_ref[...]
    R = x.shape[0]
    for w_ref, p_ref in ((w1_ref, p1_ref), (w2_ref, p2_ref), (w3_ref, p3_ref)):
        y = jnp.dot(x, w_ref[...], preferred_element_type=jnp.float32) + p_ref[0:1, :]
        mean = jnp.sum(y, axis=0, keepdims=True) / R
        d = y - mean
        var = jnp.sum(d * d, axis=0, keepdims=True) / R
        x = jnp.maximum(d / jnp.sqrt(var + 1e-5) * p_ref[1:2, :] + p_ref[2:3, :], 0.0)
    o_ref[...] = jnp.max(x.reshape(B, S, -1), axis=1)


def _sa3(x3, lp, B, S):
    C3 = lp[2][0].shape[0]
    args = []
    for (W, b, ga, be) in lp:
        args.append(W.T)
        z = jnp.zeros_like(b)
        args.append(jnp.stack([b, ga, be, z, z, z, z, z]))
    return pl.pallas_call(
        functools.partial(_sa3_body, B, S),
        out_shape=jax.ShapeDtypeStruct((B, C3), jnp.float32),
    )(x3, *args)


def kernel(points, point_features, params):
    B, _, N = points.shape
    xyz = jnp.transpose(points, (0, 2, 1))
    feats = jnp.transpose(point_features, (0, 2, 1))

    cx, cy, cz = _fps(points, 512)
    nxs1 = jnp.stack([cx, cy, cz], axis=-1)
    nxt1 = jnp.stack([cx, cy, cz], axis=1)
    idx1 = _ballq(nxs1, points, 0.2, 32)
    table1 = jnp.concatenate(
        [xyz, feats, jnp.zeros((B, N, 10), jnp.float32)], axis=-1
    ).reshape(B * N, 16)
    g1 = _sc_gather(table1, idx1.reshape(-1))
    feats1 = _mlp_sa(g1, nxs1.reshape(B * 512, 3), params[0], K=32, Dp=16, R_T=4096)

    cx2, cy2, cz2 = _fps(nxt1, 128)
    nxs2 = jnp.stack([cx2, cy2, cz2], axis=-1)
    idx2 = _ballq(nxs2, nxt1, 0.4, 64)
    table2 = jnp.concatenate(
        [
            nxs1,
            feats1.reshape(B, 512, 128),
            jnp.zeros((B, 512, 13), jnp.float32),
        ],
        axis=-1,
    ).reshape(B * 512, 144)
    g2 = _sc_gather(table2, idx2.reshape(-1))
    feats2 = _mlp_sa(g2, nxs2.reshape(B * 128, 3), params[1], K=64, Dp=144, R_T=2048)

    x3 = jnp.concatenate([nxs2.reshape(B * 128, 3), feats2], axis=-1)
    return _sa3(x3, params[2], B, 128)


# speedup vs baseline: 11.1793x; 1.3343x over previous
import functools

import jax
import jax.numpy as jnp
from jax import lax
from jax.experimental import pallas as pl
from jax.experimental.pallas import tpu as pltpu
from jax.experimental.pallas import tpu_sc as plsc


def _fps_body(npoint, N, B, xyz_ref, cx_ref, cy_ref, cz_ref):
    xr = xyz_ref[:, 0, :]
    yr = xyz_ref[:, 1, :]
    zr = xyz_ref[:, 2, :]
    lane = lax.broadcasted_iota(jnp.int32, (B, N), 1)
    col = lax.broadcasted_iota(jnp.int32, (B, npoint), 1)

    def body(i, c):
        dist, far, cx_a, cy_a, cz_a = c
        oh = lane == far
        cx = jnp.sum(jnp.where(oh, xr, 0.0), axis=1, keepdims=True)
        cy = jnp.sum(jnp.where(oh, yr, 0.0), axis=1, keepdims=True)
        cz = jnp.sum(jnp.where(oh, zr, 0.0), axis=1, keepdims=True)
        sel = col == i
        cx_a = jnp.where(sel, cx, cx_a)
        cy_a = jnp.where(sel, cy, cy_a)
        cz_a = jnp.where(sel, cz, cz_a)
        dx = xr - cx
        dy = yr - cy
        dz = zr - cz
        d = dx * dx + dy * dy + dz * dz
        dist = jnp.minimum(dist, d)
        mx = jnp.max(dist, axis=1, keepdims=True)
        far = jnp.min(jnp.where(dist == mx, lane, N), axis=1, keepdims=True)
        return (dist, far, cx_a, cy_a, cz_a)

    init = (
        jnp.full((B, N), 1e10, jnp.float32),
        jnp.zeros((B, 1), jnp.int32),
        jnp.zeros((B, npoint), jnp.float32),
        jnp.zeros((B, npoint), jnp.float32),
        jnp.zeros((B, npoint), jnp.float32),
    )
    _, _, cx_a, cy_a, cz_a = lax.fori_loop(0, npoint, body, init)
    cx_ref[...] = cx_a
    cy_ref[...] = cy_a
    cz_ref[...] = cz_a


def _fps(xyz_t, npoint):
    B, _, N = xyz_t.shape
    out = jax.ShapeDtypeStruct((B, npoint), jnp.float32)
    return pl.pallas_call(
        functools.partial(_fps_body, npoint, N, B),
        out_shape=(out, out, out),
    )(xyz_t)


def _round_bf16(x):
    u = lax.bitcast_convert_type(x, jnp.uint32)
    r = (u + 0x7FFF + ((u >> 16) & 1)) & jnp.uint32(0xFFFF0000)
    return lax.bitcast_convert_type(r, jnp.float32)


def _bq_body(r2, K, N, S_T, nxs_ref, xyz_ref, out_ref):
    b = pl.program_id(0)
    sx = nxs_ref[0, :, 0:1]
    sy = nxs_ref[0, :, 1:2]
    sz = nxs_ref[0, :, 2:3]
    nx = xyz_ref[0, 0:1, :]
    ny = xyz_ref[0, 1:2, :]
    nz = xyz_ref[0, 2:3, :]
    dots = (
        _round_bf16(sx) * _round_bf16(nx)
        + _round_bf16(sy) * _round_bf16(ny)
        + _round_bf16(sz) * _round_bf16(nz)
    )
    s2 = sx * sx + sy * sy + sz * sz
    n2 = nx * nx + ny * ny + nz * nz
    sqd = (s2 + n2) - 2.0 * dots
    lane = lax.broadcasted_iota(jnp.int32, (S_T, N), 1)
    colk = lax.broadcasted_iota(jnp.int32, (S_T, K), 1)
    sent = jnp.int32(N)
    masked = jnp.where(sqd <= r2, lane, sent)
    first = jnp.min(masked, axis=1, keepdims=True)

    def step(j, c):
        m, out = c
        sel = jnp.where(m == sent, first, m)
        out = jnp.where(colk == j, sel, out)
        m = jnp.min(jnp.where(masked > m, masked, sent), axis=1, keepdims=True)
        return (m, out)

    _, out = lax.fori_loop(0, K, step, (first, jnp.zeros((S_T, K), jnp.int32)))
    out = jnp.minimum(out, N - 1)
    out_ref[...] = (out + b * N)[None, :, :]


def _ballq(nxs, xyz_t, radius, K):
    B, S, _ = nxs.shape
    N = xyz_t.shape[2]
    S_T = 64
    return pl.pallas_call(
        functools.partial(_bq_body, radius * radius, K, N, S_T),
        grid=(B, S // S_T),
        in_specs=[
            pl.BlockSpec((1, S_T, 3), lambda b, s: (b, s, 0)),
            pl.BlockSpec((1, 3, N), lambda b, s: (b, 0, 0)),
        ],
        out_specs=pl.BlockSpec((1, S_T, K), lambda b, s: (b, s, 0)),
        out_shape=jax.ShapeDtypeStruct((B, S, K), jnp.int32),
    )(nxs, xyz_t)


def _sc_gather(table, idx):
    V, D = table.shape
    R = idx.shape[0]
    NW = 32
    rpw = R // NW
    nch = rpw // 128
    mesh = plsc.VectorSubcoreMesh(core_axis_name="c", subcore_axis_name="s")

    @functools.partial(
        pl.kernel,
        out_type=jax.ShapeDtypeStruct((R, D), jnp.float32),
        mesh=mesh,
        compiler_params=pltpu.CompilerParams(use_tc_tiling_on_sc=False),
        scratch_types=[
            pltpu.VMEM((rpw,), jnp.int32),
            pltpu.VMEM((128, D), jnp.float32),
            pltpu.SemaphoreType.DMA,
        ],
    )
    def k(table_hbm, idx_hbm, out_hbm, idx_v, buf_v, sem):
        wid = lax.axis_index("s") * 2 + lax.axis_index("c")
        base = wid * rpw
        pltpu.sync_copy(idx_hbm.at[pl.ds(base, rpw)], idx_v)

        def body(i, _):
            pltpu.async_copy(
                table_hbm.at[idx_v.at[pl.ds(i * 128, 128)]], buf_v, sem
            ).wait()
            pltpu.sync_copy(buf_v, out_hbm.at[pl.ds(base + i * 128, 128)])
            return 0

        lax.fori_loop(0, nch, body, 0)

    return k(table, idx)


def _mlpA_body(K, nxp_ref, g_ref, w_ref, bias_ref, y_ref, st_ref):
    R_T = g_ref.shape[0]
    G_T = R_T // K
    x = g_ref[...].reshape(G_T, K, -1) - nxp_ref[...][:, None, :]
    x = x.reshape(R_T, -1)
    y = jnp.dot(x, w_ref[...], preferred_element_type=jnp.float32) + bias_ref[0:1, :]
    y_ref[...] = y

    @pl.when(pl.program_id(0) == 0)
    def _():
        st_ref[...] = jnp.zeros_like(st_ref)

    st_ref[0:1, :] += jnp.sum(y, axis=0, keepdims=True)
    st_ref[1:2, :] += jnp.sum(y * y, axis=0, keepdims=True)


def _mlpB_body(aff_ref, y_ref, w_ref, bias_ref, y2_ref, st_ref):
    z = jnp.maximum(y_ref[...] * aff_ref[0:1, :] + aff_ref[1:2, :], 0.0)
    y2 = jnp.dot(z, w_ref[...], preferred_element_type=jnp.float32) + bias_ref[0:1, :]
    y2_ref[...] = y2

    @pl.when(pl.program_id(0) == 0)
    def _():
        st_ref[...] = jnp.zeros_like(st_ref)

    st_ref[0:1, :] += jnp.sum(y2, axis=0, keepdims=True)
    st_ref[1:2, :] += jnp.sum(y2 * y2, axis=0, keepdims=True)


def _mlpD_body(K, aff_ref, y_ref, o_ref):
    z = jnp.maximum(y_ref[...] * aff_ref[0:1, :] + aff_ref[1:2, :], 0.0)
    R_T = z.shape[0]
    o_ref[...] = jnp.max(z.reshape(R_T // K, K, -1), axis=1)


def _affine(st, gamma, beta, count):
    mean = st[0] / count
    var = st[1] / count - mean * mean
    scale = gamma / jnp.sqrt(var + 1e-5)
    shift = beta - mean * scale
    z = jnp.zeros_like(scale)
    return jnp.stack([scale, shift, z, z, z, z, z, z])


def _row8(v):
    return jnp.concatenate([v[None, :], jnp.zeros((7, v.shape[0]), v.dtype)])


def _mlp_sa(g, nxs, lp, K, Dp, R_T):
    R = g.shape[0]
    G = R // K
    G_T = R_T // K
    grid = (R // R_T,)
    (W1, b1, ga1, be1), (W2, b2, ga2, be2), (W3, b3, ga3, be3) = lp
    C1, C2, C3 = W1.shape[0], W2.shape[0], W3.shape[0]
    cnt = jnp.float32(R)

    W1p = jnp.zeros((Dp, C1), jnp.float32).at[: W1.shape[1], :].set(W1.T)
    nxp = jnp.zeros((G, Dp), jnp.float32).at[:, :3].set(nxs)

    def statspec(C):
        return pl.BlockSpec((8, C), lambda i: (0, 0))

    y1, st1 = pl.pallas_call(
        functools.partial(_mlpA_body, K),
        grid=grid,
        in_specs=[
            pl.BlockSpec((G_T, Dp), lambda i: (i, 0)),
            pl.BlockSpec((R_T, Dp), lambda i: (i, 0)),
            pl.BlockSpec((Dp, C1), lambda i: (0, 0)),
            statspec(C1),
        ],
        out_specs=[pl.BlockSpec((R_T, C1), lambda i: (i, 0)), statspec(C1)],
        out_shape=[
            jax.ShapeDtypeStruct((R, C1), jnp.float32),
            jax.ShapeDtypeStruct((8, C1), jnp.float32),
        ],
    )(nxp, g, W1p, _row8(b1))

    def stage_b(aff, y, W, b, Cin, Cout):
        return pl.pallas_call(
            _mlpB_body,
            grid=grid,
            in_specs=[
                statspec(Cin),
                pl.BlockSpec((R_T, Cin), lambda i: (i, 0)),
                pl.BlockSpec((Cin, Cout), lambda i: (0, 0)),
                statspec(Cout),
            ],
            out_specs=[pl.BlockSpec((R_T, Cout), lambda i: (i, 0)), statspec(Cout)],
            out_shape=[
                jax.ShapeDtypeStruct((R, Cout), jnp.float32),
                jax.ShapeDtypeStruct((8, Cout), jnp.float32),
            ],
        )(aff, y, W.T, _row8(b))

    y2, st2 = stage_b(_affine(st1, ga1, be1, cnt), y1, W2, b2, C1, C2)
    y3, st3 = stage_b(_affine(st2, ga2, be2, cnt), y2, W3, b3, C2, C3)

    out = pl.pallas_call(
        functools.partial(_mlpD_body, K),
        grid=grid,
        in_specs=[
            statspec(C3),
            pl.BlockSpec((R_T, C3), lambda i: (i, 0)),
        ],
        out_specs=pl.BlockSpec((G_T, C3), lambda i: (i, 0)),
        out_shape=jax.ShapeDtypeStruct((G, C3), jnp.float32),
    )(_affine(st3, ga3, be3, cnt), y3)
    return out


def _sa3_body(B, S, x_ref, w1_ref, p1_ref, w2_ref, p2_ref, w3_ref, p3_ref, o_ref):
    x = x_ref[...]
    R = x.shape[0]
    for w_ref, p_ref in ((w1_ref, p1_ref), (w2_ref, p2_ref), (w3_ref, p3_ref)):
        y = jnp.dot(x, w_ref[...], preferred_element_type=jnp.float32) + p_ref[0:1, :]
        mean = jnp.sum(y, axis=0, keepdims=True) / R
        d = y - mean
        var = jnp.sum(d * d, axis=0, keepdims=True) / R
        x = jnp.maximum(d / jnp.sqrt(var + 1e-5) * p_ref[1:2, :] + p_ref[2:3, :], 0.0)
    o_ref[...] = jnp.max(x.reshape(B, S, -1), axis=1)


def _sa3(x3, lp, B, S):
    C3 = lp[2][0].shape[0]
    args = []
    for (W, b, ga, be) in lp:
        args.append(W.T)
        z = jnp.zeros_like(b)
        args.append(jnp.stack([b, ga, be, z, z, z, z, z]))
    return pl.pallas_call(
        functools.partial(_sa3_body, B, S),
        out_shape=jax.ShapeDtypeStruct((B, C3), jnp.float32),
    )(x3, *args)


def kernel(points, point_features, params):
    B, _, N = points.shape
    xyz = jnp.transpose(points, (0, 2, 1))
    feats = jnp.transpose(point_features, (0, 2, 1))

    cx, cy, cz = _fps(points, 512)
    nxs1 = jnp.stack([cx, cy, cz], axis=-1)
    nxt1 = jnp.stack([cx, cy, cz], axis=1)
    idx1 = _ballq(nxs1, points, 0.2, 32)
    table1 = jnp.concatenate(
        [xyz, feats, jnp.zeros((B, N, 10), jnp.float32)], axis=-1
    ).reshape(B * N, 16)
    g1 = _sc_gather(table1, idx1.reshape(-1))
    feats1 = _mlp_sa(g1, nxs1.reshape(B * 512, 3), params[0], K=32, Dp=16, R_T=4096)

    cx2, cy2, cz2 = _fps(nxt1, 128)
    nxs2 = jnp.stack([cx2, cy2, cz2], axis=-1)
    idx2 = _ballq(nxs2, nxt1, 0.4, 64)
    table2 = jnp.concatenate(
        [
            nxs1,
            feats1.reshape(B, 512, 128),
            jnp.zeros((B, 512, 13), jnp.float32),
        ],
        axis=-1,
    ).reshape(B * 512, 144)
    g2 = _sc_gather(table2, idx2.reshape(-1))
    feats2 = _mlp_sa(g2, nxs2.reshape(B * 128, 3), params[1], K=64, Dp=144, R_T=2048)

    x3 = jnp.concatenate([nxs2.reshape(B * 128, 3), feats2], axis=-1)
    return _sa3(x3, params[2], B, 128)


# speedup vs baseline: 16.0606x; 1.4366x over previous
import functools

import jax
import jax.numpy as jnp
from jax import lax
from jax.experimental import pallas as pl
from jax.experimental.pallas import tpu as pltpu
from jax.experimental.pallas import tpu_sc as plsc


def _fps_body(npoint, N, B, xyz_ref, cx_ref, cy_ref, cz_ref):
    xr = xyz_ref[:, 0, :]
    yr = xyz_ref[:, 1, :]
    zr = xyz_ref[:, 2, :]
    lane = lax.broadcasted_iota(jnp.int32, (B, N), 1)
    col = lax.broadcasted_iota(jnp.int32, (B, npoint), 1)

    def body(i, c):
        dist, far, cx_a, cy_a, cz_a = c
        oh = lane == far
        cx = jnp.sum(jnp.where(oh, xr, 0.0), axis=1, keepdims=True)
        cy = jnp.sum(jnp.where(oh, yr, 0.0), axis=1, keepdims=True)
        cz = jnp.sum(jnp.where(oh, zr, 0.0), axis=1, keepdims=True)
        sel = col == i
        cx_a = jnp.where(sel, cx, cx_a)
        cy_a = jnp.where(sel, cy, cy_a)
        cz_a = jnp.where(sel, cz, cz_a)
        dx = xr - cx
        dy = yr - cy
        dz = zr - cz
        d = dx * dx + dy * dy + dz * dz
        dist = jnp.minimum(dist, d)
        mx = jnp.max(dist, axis=1, keepdims=True)
        far = jnp.min(jnp.where(dist == mx, lane, N), axis=1, keepdims=True)
        return (dist, far, cx_a, cy_a, cz_a)

    init = (
        jnp.full((B, N), 1e10, jnp.float32),
        jnp.zeros((B, 1), jnp.int32),
        jnp.zeros((B, npoint), jnp.float32),
        jnp.zeros((B, npoint), jnp.float32),
        jnp.zeros((B, npoint), jnp.float32),
    )
    _, _, cx_a, cy_a, cz_a = lax.fori_loop(0, npoint, body, init)
    cx_ref[...] = cx_a
    cy_ref[...] = cy_a
    cz_ref[...] = cz_a


def _fps(xyz_t, npoint):
    B, _, N = xyz_t.shape
    out = jax.ShapeDtypeStruct((B, npoint), jnp.float32)
    return pl.pallas_call(
        functools.partial(_fps_body, npoint, N, B),
        out_shape=(out, out, out),
    )(xyz_t)


def _round_bf16(x):
    u = lax.bitcast_convert_type(x, jnp.uint32)
    r = (u + 0x7FFF + ((u >> 16) & 1)) & jnp.uint32(0xFFFF0000)
    return lax.bitcast_convert_type(r, jnp.float32)


def _bq_body(r2, K, N, S_T, nxs_ref, xyz_ref, out_ref):
    b = pl.program_id(0)
    sx = nxs_ref[0, :, 0:1]
    sy = nxs_ref[0, :, 1:2]
    sz = nxs_ref[0, :, 2:3]
    nx = xyz_ref[0, 0:1, :]
    ny = xyz_ref[0, 1:2, :]
    nz = xyz_ref[0, 2:3, :]
    dots = (
        _round_bf16(sx) * _round_bf16(nx)
        + _round_bf16(sy) * _round_bf16(ny)
        + _round_bf16(sz) * _round_bf16(nz)
    )
    s2 = sx * sx + sy * sy + sz * sz
    n2 = nx * nx + ny * ny + nz * nz
    sqd = (s2 + n2) - 2.0 * dots
    lane = lax.broadcasted_iota(jnp.int32, (S_T, N), 1)
    colk = lax.broadcasted_iota(jnp.int32, (S_T, K), 1)
    sent = jnp.int32(N)
    masked = jnp.where(sqd <= r2, lane, sent)
    first = jnp.min(masked, axis=1, keepdims=True)

    def cond(c):
        j, m, _ = c
        return (j < K) & (jnp.min(m) < sent)

    def step(c):
        j, m, out = c
        sel = jnp.where(m == sent, first, m)
        out = jnp.where(colk == j, sel, out)
        m = jnp.min(jnp.where(masked > m, masked, sent), axis=1, keepdims=True)
        return (j + 1, m, out)

    jend, _, out = lax.while_loop(
        cond, step, (0, first, jnp.zeros((S_T, K), jnp.int32))
    )
    out = jnp.where(colk >= jend, first, out)
    out = jnp.minimum(out, N - 1)
    out_ref[...] = (out + b * N)[None, :, :]


def _ballq(nxs, xyz_t, radius, K):
    B, S, _ = nxs.shape
    N = xyz_t.shape[2]
    S_T = 128 if S % 128 == 0 else 64
    return pl.pallas_call(
        functools.partial(_bq_body, radius * radius, K, N, S_T),
        grid=(B, S // S_T),
        in_specs=[
            pl.BlockSpec((1, S_T, 3), lambda b, s: (b, s, 0)),
            pl.BlockSpec((1, 3, N), lambda b, s: (b, 0, 0)),
        ],
        out_specs=pl.BlockSpec((1, S_T, K), lambda b, s: (b, s, 0)),
        out_shape=jax.ShapeDtypeStruct((B, S, K), jnp.int32),
    )(nxs, xyz_t)


def _sc_gather(table, idx):
    V, D = table.shape
    R = idx.shape[0]
    NW = 32
    rpw = R // NW
    nch = rpw // 128
    mesh = plsc.VectorSubcoreMesh(core_axis_name="c", subcore_axis_name="s")

    @functools.partial(
        pl.kernel,
        out_type=jax.ShapeDtypeStruct((R, D), jnp.float32),
        mesh=mesh,
        compiler_params=pltpu.CompilerParams(use_tc_tiling_on_sc=False),
        scratch_types=[
            pltpu.VMEM((rpw,), jnp.int32),
            pltpu.VMEM((128, D), jnp.float32),
            pltpu.SemaphoreType.DMA,
        ],
    )
    def k(table_hbm, idx_hbm, out_hbm, idx_v, buf_v, sem):
        wid = lax.axis_index("s") * 2 + lax.axis_index("c")
        base = wid * rpw
        pltpu.sync_copy(idx_hbm.at[pl.ds(base, rpw)], idx_v)

        def body(i, _):
            pltpu.async_copy(
                table_hbm.at[idx_v.at[pl.ds(i * 128, 128)]], buf_v, sem
            ).wait()
            pltpu.sync_copy(buf_v, out_hbm.at[pl.ds(base + i * 128, 128)])
            return 0

        lax.fori_loop(0, nch, body, 0)

    return k(table, idx)


def _mlpA_body(K, nxp_ref, g_ref, w_ref, bias_ref, y_ref, st_ref):
    R_T = g_ref.shape[0]
    G_T = R_T // K
    x = g_ref[...].reshape(G_T, K, -1) - nxp_ref[...][:, None, :]
    x = x.reshape(R_T, -1)
    y = jnp.dot(x, w_ref[...], preferred_element_type=jnp.float32) + bias_ref[0:1, :]
    y_ref[...] = y

    @pl.when(pl.program_id(0) == 0)
    def _():
        st_ref[...] = jnp.zeros_like(st_ref)

    st_ref[0:1, :] += jnp.sum(y, axis=0, keepdims=True)
    st_ref[1:2, :] += jnp.sum(y * y, axis=0, keepdims=True)


def _mlpB_body(aff_ref, y_ref, w_ref, bias_ref, y2_ref, st_ref):
    z = jnp.maximum(y_ref[...] * aff_ref[0:1, :] + aff_ref[1:2, :], 0.0)
    y2 = jnp.dot(z, w_ref[...], preferred_element_type=jnp.float32) + bias_ref[0:1, :]
    y2_ref[...] = y2

    @pl.when(pl.program_id(0) == 0)
    def _():
        st_ref[...] = jnp.zeros_like(st_ref)

    st_ref[0:1, :] += jnp.sum(y2, axis=0, keepdims=True)
    st_ref[1:2, :] += jnp.sum(y2 * y2, axis=0, keepdims=True)


def _mlpD_body(K, aff_ref, y_ref, o_ref):
    z = jnp.maximum(y_ref[...] * aff_ref[0:1, :] + aff_ref[1:2, :], 0.0)
    R_T = z.shape[0]
    o_ref[...] = jnp.max(z.reshape(R_T // K, K, -1), axis=1)


def _affine(st, gamma, beta, count):
    mean = st[0] / count
    var = st[1] / count - mean * mean
    scale = gamma / jnp.sqrt(var + 1e-5)
    shift = beta - mean * scale
    z = jnp.zeros_like(scale)
    return jnp.stack([scale, shift, z, z, z, z, z, z])


def _row8(v):
    return jnp.concatenate([v[None, :], jnp.zeros((7, v.shape[0]), v.dtype)])


def _mlp_sa(g, nxs, lp, K, Dp, R_T):
    R = g.shape[0]
    G = R // K
    G_T = R_T // K
    grid = (R // R_T,)
    (W1, b1, ga1, be1), (W2, b2, ga2, be2), (W3, b3, ga3, be3) = lp
    C1, C2, C3 = W1.shape[0], W2.shape[0], W3.shape[0]
    cnt = jnp.float32(R)

    W1p = jnp.zeros((Dp, C1), jnp.float32).at[: W1.shape[1], :].set(W1.T)
    nxp = jnp.zeros((G, Dp), jnp.float32).at[:, :3].set(nxs)

    def statspec(C):
        return pl.BlockSpec((8, C), lambda i: (0, 0))

    y1, st1 = pl.pallas_call(
        functools.partial(_mlpA_body, K),
        grid=grid,
        in_specs=[
            pl.BlockSpec((G_T, Dp), lambda i: (i, 0)),
            pl.BlockSpec((R_T, Dp), lambda i: (i, 0)),
            pl.BlockSpec((Dp, C1), lambda i: (0, 0)),
            statspec(C1),
        ],
        out_specs=[pl.BlockSpec((R_T, C1), lambda i: (i, 0)), statspec(C1)],
        out_shape=[
            jax.ShapeDtypeStruct((R, C1), jnp.float32),
            jax.ShapeDtypeStruct((8, C1), jnp.float32),
        ],
    )(nxp, g, W1p, _row8(b1))

    def stage_b(aff, y, W, b, Cin, Cout):
        return pl.pallas_call(
            _mlpB_body,
            grid=grid,
            in_specs=[
                statspec(Cin),
                pl.BlockSpec((R_T, Cin), lambda i: (i, 0)),
                pl.BlockSpec((Cin, Cout), lambda i: (0, 0)),
                statspec(Cout),
            ],
            out_specs=[pl.BlockSpec((R_T, Cout), lambda i: (i, 0)), statspec(Cout)],
            out_shape=[
                jax.ShapeDtypeStruct((R, Cout), jnp.float32),
                jax.ShapeDtypeStruct((8, Cout), jnp.float32),
            ],
        )(aff, y, W.T, _row8(b))

    y2, st2 = stage_b(_affine(st1, ga1, be1, cnt), y1, W2, b2, C1, C2)
    y3, st3 = stage_b(_affine(st2, ga2, be2, cnt), y2, W3, b3, C2, C3)

    out = pl.pallas_call(
        functools.partial(_mlpD_body, K),
        grid=grid,
        in_specs=[
            statspec(C3),
            pl.BlockSpec((R_T, C3), lambda i: (i, 0)),
        ],
        out_specs=pl.BlockSpec((G_T, C3), lambda i: (i, 0)),
        out_shape=jax.ShapeDtypeStruct((G, C3), jnp.float32),
    )(_affine(st3, ga3, be3, cnt), y3)
    return out


def _sa3_body(B, S, x_ref, w1_ref, p1_ref, w2_ref, p2_ref, w3_ref, p3_ref, o_ref):
    x = x_ref[...]
    R = x.shape[0]
    for w_ref, p_ref in ((w1_ref, p1_ref), (w2_ref, p2_ref), (w3_ref, p3_ref)):
        y = jnp.dot(x, w_ref[...], preferred_element_type=jnp.float32) + p_ref[0:1, :]
        mean = jnp.sum(y, axis=0, keepdims=True) / R
        d = y - mean
        var = jnp.sum(d * d, axis=0, keepdims=True) / R
        x = jnp.maximum(d / jnp.sqrt(var + 1e-5) * p_ref[1:2, :] + p_ref[2:3, :], 0.0)
    o_ref[...] = jnp.max(x.reshape(B, S, -1), axis=1)


def _sa3(x3, lp, B, S):
    C3 = lp[2][0].shape[0]
    args = []
    for (W, b, ga, be) in lp:
        args.append(W.T)
        z = jnp.zeros_like(b)
        args.append(jnp.stack([b, ga, be, z, z, z, z, z]))
    return pl.pallas_call(
        functools.partial(_sa3_body, B, S),
        out_shape=jax.ShapeDtypeStruct((B, C3), jnp.float32),
    )(x3, *args)


def kernel(points, point_features, params):
    B, _, N = points.shape
    xyz = jnp.transpose(points, (0, 2, 1))
    feats = jnp.transpose(point_features, (0, 2, 1))

    cx, cy, cz = _fps(points, 512)
    nxs1 = jnp.stack([cx, cy, cz], axis=-1)
    nxt1 = jnp.stack([cx, cy, cz], axis=1)
    idx1 = _ballq(nxs1, points, 0.2, 32)
    table1 = jnp.concatenate(
        [xyz, feats, jnp.zeros((B, N, 10), jnp.float32)], axis=-1
    ).reshape(B * N, 16)
    g1 = _sc_gather(table1, idx1.reshape(-1))
    feats1 = _mlp_sa(g1, nxs1.reshape(B * 512, 3), params[0], K=32, Dp=16, R_T=4096)

    cx2, cy2, cz2 = _fps(nxt1, 128)
    nxs2 = jnp.stack([cx2, cy2, cz2], axis=-1)
    idx2 = _ballq(nxs2, nxt1, 0.4, 64)
    table2 = jnp.concatenate(
        [
            nxs1,
            feats1.reshape(B, 512, 128),
            jnp.zeros((B, 512, 13), jnp.float32),
        ],
        axis=-1,
    ).reshape(B * 512, 144)
    g2 = _sc_gather(table2, idx2.reshape(-1))
    feats2 = _mlp_sa(g2, nxs2.reshape(B * 128, 3), params[1], K=64, Dp=144, R_T=2048)

    x3 = jnp.concatenate([nxs2.reshape(B * 128, 3), feats2], axis=-1)
    return _sa3(x3, params[2], B, 128)


# speedup vs baseline: 16.8779x; 1.0509x over previous
import functools

import jax
import jax.numpy as jnp
from jax import lax
from jax.experimental import pallas as pl
from jax.experimental.pallas import tpu as pltpu
from jax.experimental.pallas import tpu_sc as plsc


def _fps_body(npoint, N, B, xyz_ref, cx_ref, cy_ref, cz_ref):
    xr = xyz_ref[:, 0, :]
    yr = xyz_ref[:, 1, :]
    zr = xyz_ref[:, 2, :]
    lane = lax.broadcasted_iota(jnp.int32, (B, N), 1)
    col = lax.broadcasted_iota(jnp.int32, (B, npoint), 1)

    def body(i, c):
        dist, far, cx_a, cy_a, cz_a = c
        oh = lane == far
        cx = jnp.sum(jnp.where(oh, xr, 0.0), axis=1, keepdims=True)
        cy = jnp.sum(jnp.where(oh, yr, 0.0), axis=1, keepdims=True)
        cz = jnp.sum(jnp.where(oh, zr, 0.0), axis=1, keepdims=True)
        sel = col == i
        cx_a = jnp.where(sel, cx, cx_a)
        cy_a = jnp.where(sel, cy, cy_a)
        cz_a = jnp.where(sel, cz, cz_a)
        dx = xr - cx
        dy = yr - cy
        dz = zr - cz
        d = dx * dx + dy * dy + dz * dz
        dist = jnp.minimum(dist, d)
        far = jnp.argmax(dist, axis=1).astype(jnp.int32).reshape(B, 1)
        return (dist, far, cx_a, cy_a, cz_a)

    init = (
        jnp.full((B, N), 1e10, jnp.float32),
        jnp.zeros((B, 1), jnp.int32),
        jnp.zeros((B, npoint), jnp.float32),
        jnp.zeros((B, npoint), jnp.float32),
        jnp.zeros((B, npoint), jnp.float32),
    )
    _, _, cx_a, cy_a, cz_a = lax.fori_loop(0, npoint, body, init)
    cx_ref[...] = cx_a
    cy_ref[...] = cy_a
    cz_ref[...] = cz_a


def _fps(xyz_t, npoint):
    B, _, N = xyz_t.shape
    out = jax.ShapeDtypeStruct((B, npoint), jnp.float32)
    return pl.pallas_call(
        functools.partial(_fps_body, npoint, N, B),
        out_shape=(out, out, out),
    )(xyz_t)


def _round_bf16(x):
    u = lax.bitcast_convert_type(x, jnp.uint32)
    r = (u + 0x7FFF + ((u >> 16) & 1)) & jnp.uint32(0xFFFF0000)
    return lax.bitcast_convert_type(r, jnp.float32)


def _bq_body(r2, K, N, S_T, nxs_ref, xyz_ref, out_ref):
    b = pl.program_id(0)
    sx = nxs_ref[0, :, 0:1]
    sy = nxs_ref[0, :, 1:2]
    sz = nxs_ref[0, :, 2:3]
    nx = xyz_ref[0, 0:1, :]
    ny = xyz_ref[0, 1:2, :]
    nz = xyz_ref[0, 2:3, :]
    dots = (
        _round_bf16(sx) * _round_bf16(nx)
        + _round_bf16(sy) * _round_bf16(ny)
        + _round_bf16(sz) * _round_bf16(nz)
    )
    s2 = sx * sx + sy * sy + sz * sz
    n2 = nx * nx + ny * ny + nz * nz
    sqd = (s2 + n2) - 2.0 * dots
    lane = lax.broadcasted_iota(jnp.int32, (S_T, N), 1)
    colk = lax.broadcasted_iota(jnp.int32, (S_T, K), 1)
    sent = jnp.int32(N)
    masked = jnp.where(sqd <= r2, lane, sent)
    first = jnp.min(masked, axis=1, keepdims=True)

    def cond(c):
        j, m, _ = c
        return (j < K) & (jnp.min(m) < sent)

    def step(c):
        j, m, out = c
        sel = jnp.where(m == sent, first, m)
        out = jnp.where(colk == j, sel, out)
        m = jnp.min(jnp.where(masked > m, masked, sent), axis=1, keepdims=True)
        return (j + 1, m, out)

    jend, _, out = lax.while_loop(
        cond, step, (0, first, jnp.zeros((S_T, K), jnp.int32))
    )
    out = jnp.where(colk >= jend, first, out)
    out = jnp.minimum(out, N - 1)
    out_ref[...] = (out + b * N)[None, :, :]


def _ballq(nxs, xyz_t, radius, K):
    B, S, _ = nxs.shape
    N = xyz_t.shape[2]
    S_T = 128 if S % 128 == 0 else 64
    return pl.pallas_call(
        functools.partial(_bq_body, radius * radius, K, N, S_T),
        grid=(B, S // S_T),
        in_specs=[
            pl.BlockSpec((1, S_T, 3), lambda b, s: (b, s, 0)),
            pl.BlockSpec((1, 3, N), lambda b, s: (b, 0, 0)),
        ],
        out_specs=pl.BlockSpec((1, S_T, K), lambda b, s: (b, s, 0)),
        out_shape=jax.ShapeDtypeStruct((B, S, K), jnp.int32),
    )(nxs, xyz_t)


def _sc_gather(table, idx):
    V, D = table.shape
    R = idx.shape[0]
    NW = 32
    rpw = R // NW
    nch = rpw // 128
    mesh = plsc.VectorSubcoreMesh(core_axis_name="c", subcore_axis_name="s")

    @functools.partial(
        pl.kernel,
        out_type=jax.ShapeDtypeStruct((R, D), jnp.float32),
        mesh=mesh,
        compiler_params=pltpu.CompilerParams(use_tc_tiling_on_sc=False),
        scratch_types=[
            pltpu.VMEM((rpw,), jnp.int32),
            pltpu.VMEM((128, D), jnp.float32),
            pltpu.VMEM((128, D), jnp.float32),
            pltpu.SemaphoreType.DMA,
            pltpu.SemaphoreType.DMA,
        ],
    )
    def k(table_hbm, idx_hbm, out_hbm, idx_v, buf0, buf1, sem0, sem1):
        wid = lax.axis_index("s") * 2 + lax.axis_index("c")
        base = wid * rpw
        pltpu.sync_copy(idx_hbm.at[pl.ds(base, rpw)], idx_v)

        def gather(ch, buf, sem):
            pltpu.async_copy(
                table_hbm.at[idx_v.at[pl.ds(ch * 128, 128)]], buf, sem
            )

        def wait_gather(ch, buf, sem):
            pltpu.make_async_copy(
                table_hbm.at[idx_v.at[pl.ds(ch * 128, 128)]], buf, sem
            ).wait()

        gather(0, buf0, sem0)
        gather(1, buf1, sem1)

        def body(i, _):
            g = i * 2
            wait_gather(g, buf0, sem0)
            pltpu.sync_copy(buf0, out_hbm.at[pl.ds(base + g * 128, 128)])

            @pl.when(g + 2 < nch)
            def _():
                gather(g + 2, buf0, sem0)

            wait_gather(g + 1, buf1, sem1)
            pltpu.sync_copy(buf1, out_hbm.at[pl.ds(base + (g + 1) * 128, 128)])

            @pl.when(g + 3 < nch)
            def _():
                gather(g + 3, buf1, sem1)

            return 0

        lax.fori_loop(0, nch // 2, body, 0)

    return k(table, idx)


def _mlpA_body(K, nxp_ref, g_ref, w_ref, bias_ref, y_ref, st_ref):
    R_T = g_ref.shape[0]
    G_T = R_T // K
    x = g_ref[...].reshape(G_T, K, -1) - nxp_ref[...][:, None, :]
    x = x.reshape(R_T, -1)
    y = jnp.dot(x, w_ref[...], preferred_element_type=jnp.float32) + bias_ref[0:1, :]
    y_ref[...] = y

    @pl.when(pl.program_id(0) == 0)
    def _():
        st_ref[...] = jnp.zeros_like(st_ref)

    st_ref[0:1, :] += jnp.sum(y, axis=0, keepdims=True)
    st_ref[1:2, :] += jnp.sum(y * y, axis=0, keepdims=True)


def _mlpB_body(aff_ref, y_ref, w_ref, bias_ref, y2_ref, st_ref):
    z = jnp.maximum(y_ref[...] * aff_ref[0:1, :] + aff_ref[1:2, :], 0.0)
    y2 = jnp.dot(z, w_ref[...], preferred_element_type=jnp.float32) + bias_ref[0:1, :]
    y2_ref[...] = y2

    @pl.when(pl.program_id(0) == 0)
    def _():
        st_ref[...] = jnp.zeros_like(st_ref)

    st_ref[0:1, :] += jnp.sum(y2, axis=0, keepdims=True)
    st_ref[1:2, :] += jnp.sum(y2 * y2, axis=0, keepdims=True)


def _mlpD_body(K, aff_ref, y_ref, o_ref):
    z = jnp.maximum(y_ref[...] * aff_ref[0:1, :] + aff_ref[1:2, :], 0.0)
    R_T = z.shape[0]
    o_ref[...] = jnp.max(z.reshape(R_T // K, K, -1), axis=1)


def _affine(st, gamma, beta, count):
    mean = st[0] / count
    var = st[1] / count - mean * mean
    scale = gamma / jnp.sqrt(var + 1e-5)
    shift = beta - mean * scale
    z = jnp.zeros_like(scale)
    return jnp.stack([scale, shift, z, z, z, z, z, z])


def _row8(v):
    return jnp.concatenate([v[None, :], jnp.zeros((7, v.shape[0]), v.dtype)])


def _mlp_sa(g, nxs, lp, K, Dp, R_T):
    R = g.shape[0]
    G = R // K
    G_T = R_T // K
    grid = (R // R_T,)
    (W1, b1, ga1, be1), (W2, b2, ga2, be2), (W3, b3, ga3, be3) = lp
    C1, C2, C3 = W1.shape[0], W2.shape[0], W3.shape[0]
    cnt = jnp.float32(R)

    W1p = jnp.zeros((Dp, C1), jnp.float32).at[: W1.shape[1], :].set(W1.T)
    nxp = jnp.zeros((G, Dp), jnp.float32).at[:, :3].set(nxs)

    def statspec(C):
        return pl.BlockSpec((8, C), lambda i: (0, 0))

    y1, st1 = pl.pallas_call(
        functools.partial(_mlpA_body, K),
        grid=grid,
        in_specs=[
            pl.BlockSpec((G_T, Dp), lambda i: (i, 0)),
            pl.BlockSpec((R_T, Dp), lambda i: (i, 0)),
            pl.BlockSpec((Dp, C1), lambda i: (0, 0)),
            statspec(C1),
        ],
        out_specs=[pl.BlockSpec((R_T, C1), lambda i: (i, 0)), statspec(C1)],
        out_shape=[
            jax.ShapeDtypeStruct((R, C1), jnp.float32),
            jax.ShapeDtypeStruct((8, C1), jnp.float32),
        ],
    )(nxp, g, W1p, _row8(b1))

    def stage_b(aff, y, W, b, Cin, Cout):
        return pl.pallas_call(
            _mlpB_body,
            grid=grid,
            in_specs=[
                statspec(Cin),
                pl.BlockSpec((R_T, Cin), lambda i: (i, 0)),
                pl.BlockSpec((Cin, Cout), lambda i: (0, 0)),
                statspec(Cout),
            ],
            out_specs=[pl.BlockSpec((R_T, Cout), lambda i: (i, 0)), statspec(Cout)],
            out_shape=[
                jax.ShapeDtypeStruct((R, Cout), jnp.float32),
                jax.ShapeDtypeStruct((8, Cout), jnp.float32),
            ],
        )(aff, y, W.T, _row8(b))

    y2, st2 = stage_b(_affine(st1, ga1, be1, cnt), y1, W2, b2, C1, C2)
    y3, st3 = stage_b(_affine(st2, ga2, be2, cnt), y2, W3, b3, C2, C3)

    out = pl.pallas_call(
        functools.partial(_mlpD_body, K),
        grid=grid,
        in_specs=[
            statspec(C3),
            pl.BlockSpec((R_T, C3), lambda i: (i, 0)),
        ],
        out_specs=pl.BlockSpec((G_T, C3), lambda i: (i, 0)),
        out_shape=jax.ShapeDtypeStruct((G, C3), jnp.float32),
    )(_affine(st3, ga3, be3, cnt), y3)
    return out


def _sa3_body(B, S, x_ref, w1_ref, p1_ref, w2_ref, p2_ref, w3_ref, p3_ref, o_ref):
    x = x_ref[...]
    R = x.shape[0]
    for w_ref, p_ref in ((w1_ref, p1_ref), (w2_ref, p2_ref), (w3_ref, p3_ref)):
        y = jnp.dot(x, w_ref[...], preferred_element_type=jnp.float32) + p_ref[0:1, :]
        mean = jnp.sum(y, axis=0, keepdims=True) / R
        d = y - mean
        var = jnp.sum(d * d, axis=0, keepdims=True) / R
        x = jnp.maximum(d / jnp.sqrt(var + 1e-5) * p_ref[1:2, :] + p_ref[2:3, :], 0.0)
    o_ref[...] = jnp.max(x.reshape(B, S, -1), axis=1)


def _sa3(x3, lp, B, S):
    C3 = lp[2][0].shape[0]
    args = []
    for (W, b, ga, be) in lp:
        args.append(W.T)
        z = jnp.zeros_like(b)
        args.append(jnp.stack([b, ga, be, z, z, z, z, z]))
    return pl.pallas_call(
        functools.partial(_sa3_body, B, S),
        out_shape=jax.ShapeDtypeStruct((B, C3), jnp.float32),
    )(x3, *args)


def kernel(points, point_features, params):
    B, _, N = points.shape
    xyz = jnp.transpose(points, (0, 2, 1))
    feats = jnp.transpose(point_features, (0, 2, 1))

    cx, cy, cz = _fps(points, 512)
    nxs1 = jnp.stack([cx, cy, cz], axis=-1)
    nxt1 = jnp.stack([cx, cy, cz], axis=1)
    idx1 = _ballq(nxs1, points, 0.2, 32)
    table1 = jnp.concatenate(
        [xyz, feats, jnp.zeros((B, N, 10), jnp.float32)], axis=-1
    ).reshape(B * N, 16)
    g1 = _sc_gather(table1, idx1.reshape(-1))
    feats1 = _mlp_sa(g1, nxs1.reshape(B * 512, 3), params[0], K=32, Dp=16, R_T=4096)

    cx2, cy2, cz2 = _fps(nxt1, 128)
    nxs2 = jnp.stack([cx2, cy2, cz2], axis=-1)
    idx2 = _ballq(nxs2, nxt1, 0.4, 64)
    table2 = jnp.concatenate(
        [
            nxs1,
            feats1.reshape(B, 512, 128),
            jnp.zeros((B, 512, 13), jnp.float32),
        ],
        axis=-1,
    ).reshape(B * 512, 144)
    g2 = _sc_gather(table2, idx2.reshape(-1))
    feats2 = _mlp_sa(g2, nxs2.reshape(B * 128, 3), params[1], K=64, Dp=144, R_T=2048)

    x3 = jnp.concatenate([nxs2.reshape(B * 128, 3), feats2], axis=-1)
    return _sa3(x3, params[2], B, 128)
